# baseline (speedup 1.0000x reference)
import jax
import jax.numpy as jnp
from jax import lax
from jax.experimental import pallas as pl
from jax.experimental.pallas import tpu as pltpu
from jax.experimental.pallas import tpu_sc as plsc
import functools

NP = 256
H = 768
NB = 32
BI = 16
U_ROWS = 512
NC, NS = 2, 16
NW = NC * NS
ROWS_PER_W = 8 * U_ROWS // NW
CHUNK = 64


def _sc_lookup_body(tok_hbm, rh_hbm, rw_hbm, u8_hbm,
                    idx_c, idx_b, buf_h, buf_w, rows, sem):
    wid = lax.axis_index("s") * NC + lax.axis_index("c")
    r = wid // 4
    m0_w = (wid % 4) * ROWS_PER_W

    for ch in range(ROWS_PER_W // CHUNK):
        m0 = m0_w + ch * CHUNK
        g0 = r * U_ROWS + m0

        for k in range(CHUNK // 16):
            mv = m0 + k * 16 + lax.broadcasted_iota(jnp.int32, (16,), 0)
            d = jnp.abs(255 - r - mv)
            idx_c[pl.ds(k * 16, 16)] = jnp.minimum(d, NB)
            idx_b[pl.ds(k * 16, 16)] = jnp.minimum(
                lax.shift_right_logical(d, 1), NB - 1)

        cp_t = pltpu.async_copy(tok_hbm.at[idx_b], rows, sem)
        cp_h = pltpu.async_copy(rh_hbm.at[idx_c], buf_h, sem)
        cp_w = pltpu.async_copy(rw_hbm.at[idx_c], buf_w, sem)
        cp_t.wait()
        cp_h.wait()
        cp_w.wait()

        def add_row(row, _):
            for col in range(H // 2 // 16):
                sl = pl.ds(col * 16, 16)
                sr = pl.ds(H // 2 + col * 16, 16)
                rows[row, sl] = rows[row, sl] + buf_h[row, sl]
                rows[row, sr] = rows[row, sr] + buf_w[row, sl]
            return _

        lax.fori_loop(0, CHUNK, add_row, 0)

        pltpu.sync_copy(rows, u8_hbm.at[pl.ds(g0, CHUNK)])


@functools.partial(
    pl.kernel,
    out_type=jax.ShapeDtypeStruct((8 * U_ROWS, H), jnp.float32),
    mesh=plsc.VectorSubcoreMesh(core_axis_name="c", subcore_axis_name="s"),
    scratch_types=[
        pltpu.VMEM((CHUNK,), jnp.int32),
        pltpu.VMEM((CHUNK,), jnp.int32),
        pltpu.VMEM((CHUNK, H // 2), jnp.float32),
        pltpu.VMEM((CHUNK, H // 2), jnp.float32),
        pltpu.VMEM((CHUNK, H), jnp.float32),
        pltpu.SemaphoreType.DMA,
    ],
)
def _sc_lookup(*args):
    _sc_lookup_body(*args)


def _tc_broadcast(u8_ref, out_ref):
    pid = pl.program_id(0)
    for ii in range(BI):
        s = 255 - (pid * BI + ii)
        r = lax.rem(s, 8)
        q8 = pl.multiple_of(s - r, 8)
        out_ref[ii] = u8_ref[r, pl.ds(q8, NP), :]


@jax.jit
def kernel(token_embeddings, rel_height, rel_width):
    u8 = _sc_lookup(token_embeddings, rel_height, rel_width)
    u8 = u8.reshape(8, U_ROWS, H)
    return pl.pallas_call(
        _tc_broadcast,
        grid=(NP // BI,),
        in_specs=[pl.BlockSpec((8, U_ROWS, H), lambda i: (0, 0, 0))],
        out_specs=pl.BlockSpec((BI, NP, H), lambda i: (i, 0, 0)),
        out_shape=jax.ShapeDtypeStruct((NP, NP, H), jnp.float32),
    )(u8)


# speedup vs baseline: 3.1753x; 3.1753x over previous
import jax
import jax.numpy as jnp
from jax import lax
from jax.experimental import pallas as pl
from jax.experimental.pallas import tpu as pltpu
from jax.experimental.pallas import tpu_sc as plsc
import functools

NP = 256
H = 768
NB = 32
BI = 16
U_ROWS = 512
NC, NS = 2, 16
NW = NC * NS
RPW = U_ROWS // NW


def _sc_lookup_body(relhw_hbm, tok_hbm, u_hbm, idx_c, idx_b, rows_a, rows_b, sem):
    wid = lax.axis_index("s") * NC + lax.axis_index("c")
    k0 = wid * RPW

    kv = k0 + lax.broadcasted_iota(jnp.int32, (16,), 0)
    d = jnp.abs(255 - kv)
    idx_c[...] = jnp.minimum(d, NB)
    idx_b[...] = jnp.minimum(lax.shift_right_logical(d, 1), NB - 1)

    cp_a = pltpu.async_copy(relhw_hbm.at[idx_c], rows_a, sem)
    cp_b = pltpu.async_copy(tok_hbm.at[idx_b], rows_b, sem)
    cp_a.wait()
    cp_b.wait()

    for row in range(RPW):
        for col in range(H // 16):
            sl = pl.ds(col * 16, 16)
            rows_a[row, sl] = rows_a[row, sl] + rows_b[row, sl]

    pltpu.sync_copy(rows_a, u_hbm.at[pl.ds(k0, RPW)])


@functools.partial(
    pl.kernel,
    out_type=jax.ShapeDtypeStruct((U_ROWS, H), jnp.float32),
    mesh=plsc.VectorSubcoreMesh(core_axis_name="c", subcore_axis_name="s"),
    scratch_types=[
        pltpu.VMEM((RPW,), jnp.int32),
        pltpu.VMEM((RPW,), jnp.int32),
        pltpu.VMEM((RPW, H), jnp.float32),
        pltpu.VMEM((RPW, H), jnp.float32),
        pltpu.SemaphoreType.DMA,
    ],
)
def _sc_lookup(*args):
    _sc_lookup_body(*args)


def _tc_broadcast(u_ref, out_ref):
    pid = pl.program_id(0)
    for ii in range(BI):
        s = 255 - (pid * BI + ii)
        r = (7 - ii) % 8
        q8 = pl.multiple_of(s - r, 8)
        tmp = u_ref[pl.ds(q8, NP + 8), :]
        out_ref[ii] = tmp[r:r + NP]


@jax.jit
def kernel(token_embeddings, rel_height, rel_width):
    relhw = jnp.concatenate([rel_height, rel_width], axis=1)
    u = _sc_lookup(relhw, token_embeddings)
    return pl.pallas_call(
        _tc_broadcast,
        grid=(NP // BI,),
        in_specs=[pl.BlockSpec((U_ROWS, H), lambda i: (0, 0))],
        out_specs=pl.BlockSpec((BI, NP, H), lambda i: (i, 0, 0)),
        out_shape=jax.ShapeDtypeStruct((NP, NP, H), jnp.float32),
    )(u)


# speedup vs baseline: 3.2467x; 1.0225x over previous
import jax
import jax.numpy as jnp
from jax import lax
from jax.experimental import pallas as pl
from jax.experimental.pallas import tpu as pltpu
from jax.experimental.pallas import tpu_sc as plsc
import functools

NP = 256
H = 768
NB = 32
BI = 16
U_ROWS = 512
NC, NS = 2, 16
NW = NC * NS
RPW = U_ROWS // NW


def _sc_lookup_body(relhw_hbm, tok_hbm, urel_hbm, utok_hbm,
                    idx_c, idx_b, rows_a, rows_b, sem):
    wid = lax.axis_index("s") * NC + lax.axis_index("c")
    k0 = wid * RPW

    kv = k0 + lax.broadcasted_iota(jnp.int32, (16,), 0)
    d = jnp.abs(255 - kv)
    idx_c[...] = jnp.minimum(d, NB)
    idx_b[...] = jnp.minimum(lax.shift_right_logical(d, 1), NB - 1)

    cp_a = pltpu.async_copy(relhw_hbm.at[idx_c], rows_a, sem)
    cp_b = pltpu.async_copy(tok_hbm.at[idx_b], rows_b, sem)
    cp_a.wait()
    cp_b.wait()

    cp_oa = pltpu.async_copy(rows_a, urel_hbm.at[pl.ds(k0, RPW)], sem)
    cp_ob = pltpu.async_copy(rows_b, utok_hbm.at[pl.ds(k0, RPW)], sem)
    cp_oa.wait()
    cp_ob.wait()


@functools.partial(
    pl.kernel,
    out_type=(
        jax.ShapeDtypeStruct((U_ROWS, H), jnp.float32),
        jax.ShapeDtypeStruct((U_ROWS, H), jnp.float32),
    ),
    mesh=plsc.VectorSubcoreMesh(core_axis_name="c", subcore_axis_name="s"),
    scratch_types=[
        pltpu.VMEM((RPW,), jnp.int32),
        pltpu.VMEM((RPW,), jnp.int32),
        pltpu.VMEM((RPW, H), jnp.float32),
        pltpu.VMEM((RPW, H), jnp.float32),
        pltpu.SemaphoreType.DMA,
    ],
)
def _sc_lookup(*args):
    _sc_lookup_body(*args)


def _tc_broadcast(urel_ref, utok_ref, out_ref, u_ref):
    pid = pl.program_id(0)

    @pl.when(pid == 0)
    def _sum_tables():
        u_ref[...] = urel_ref[...] + utok_ref[...]

    for ii in range(BI):
        s = 255 - (pid * BI + ii)
        r = (7 - ii) % 8
        q8 = pl.multiple_of(s - r, 8)
        tmp = u_ref[pl.ds(q8, NP + 8), :]
        out_ref[ii] = tmp[r:r + NP]


@jax.jit
def kernel(token_embeddings, rel_height, rel_width):
    relhw = jnp.concatenate([rel_height, rel_width], axis=1)
    urel, utok = _sc_lookup(relhw, token_embeddings)
    return pl.pallas_call(
        _tc_broadcast,
        grid=(NP // BI,),
        in_specs=[
            pl.BlockSpec((U_ROWS, H), lambda i: (0, 0)),
            pl.BlockSpec((U_ROWS, H), lambda i: (0, 0)),
        ],
        out_specs=pl.BlockSpec((BI, NP, H), lambda i: (i, 0, 0)),
        out_shape=jax.ShapeDtypeStruct((NP, NP, H), jnp.float32),
        scratch_shapes=[pltpu.VMEM((U_ROWS, H), jnp.float32)],
    )(urel, utok)
